# baseline (device time: 1507806 ns/iter reference)
import jax
import jax.numpy as jnp
from jax import lax
from jax.experimental import pallas as pl
from jax.experimental.pallas import tpu as pltpu

N_DEV = 32
S = 3
H_R = N_DEV // 2
H_L = N_DEV - 1 - H_R


def kernel(x, w_mat):
    m_per, k = x.shape
    _, n_per = w_mat.shape
    assert m_per * N_DEV == 8192 and k == 8192

    def body(x_ref, w_ref, out_ref, xb, wb, buf_r, buf_l,
             send_r, recv_r, send_l, recv_l, credit_r, credit_l):
        me = lax.axis_index("i")
        left = lax.rem(me + N_DEV - 1, N_DEV)
        right = lax.rem(me + 1, N_DEV)

        xb[...] = x_ref[...].astype(jnp.bfloat16)
        wb[...] = w_ref[...].astype(jnp.bfloat16)

        barrier = pltpu.get_barrier_semaphore()
        pl.semaphore_signal(barrier, inc=1, device_id=(left,),
                            device_id_type=pl.DeviceIdType.MESH)
        pl.semaphore_signal(barrier, inc=1, device_id=(right,),
                            device_id_type=pl.DeviceIdType.MESH)
        pl.semaphore_wait(barrier, 2)

        def start_send(bufs, send_sems, recv_sems, dst_dev, h):
            s = (h - 1) % S
            src = xb if h == 1 else bufs.at[(h - 2) % S]
            desc = pltpu.make_async_remote_copy(
                src_ref=src,
                dst_ref=bufs.at[s],
                send_sem=send_sems.at[s],
                recv_sem=recv_sems.at[s],
                device_id=(dst_dev,),
                device_id_type=pl.DeviceIdType.MESH,
            )
            desc.start()
            return desc

        def gemm_store(chunk, origin):
            y = jnp.dot(chunk, wb[...], preferred_element_type=jnp.float32)
            out_ref[pl.ds(origin * m_per, m_per), :] = y * jax.nn.sigmoid(y)

        for h in range(1, H_R + 1):
            s = (h - 1) % S
            if h > S:
                pl.semaphore_wait(credit_r, 1)
            desc_r = start_send(buf_r, send_r, recv_r, right, h)
            desc_l = None
            if h <= H_L:
                if h > S:
                    pl.semaphore_wait(credit_l, 1)
                desc_l = start_send(buf_l, send_l, recv_l, left, h)
            if h == 1:
                gemm_store(xb[...], me)

            desc_r.wait_send()
            desc_r.wait_recv()
            if 2 <= h <= H_R - S + 1:
                pl.semaphore_signal(credit_r, inc=1, device_id=(left,),
                                    device_id_type=pl.DeviceIdType.MESH)
            gemm_store(buf_r[s], lax.rem(me + N_DEV - h, N_DEV))

            if desc_l is not None:
                desc_l.wait_send()
                desc_l.wait_recv()
                if 2 <= h <= H_L - S + 1:
                    pl.semaphore_signal(credit_l, inc=1, device_id=(right,),
                                        device_id_type=pl.DeviceIdType.MESH)
                gemm_store(buf_l[s], lax.rem(me + h, N_DEV))

    return pl.pallas_call(
        body,
        out_shape=jax.ShapeDtypeStruct((N_DEV * m_per, n_per), jnp.float32),
        in_specs=[pl.BlockSpec(memory_space=pltpu.VMEM),
                  pl.BlockSpec(memory_space=pltpu.VMEM)],
        out_specs=pl.BlockSpec(memory_space=pltpu.VMEM),
        scratch_shapes=[
            pltpu.VMEM((m_per, k), jnp.bfloat16),
            pltpu.VMEM((k, n_per), jnp.bfloat16),
            pltpu.VMEM((S, m_per, k), jnp.bfloat16),
            pltpu.VMEM((S, m_per, k), jnp.bfloat16),
            pltpu.SemaphoreType.DMA((S,)),
            pltpu.SemaphoreType.DMA((S,)),
            pltpu.SemaphoreType.DMA((S,)),
            pltpu.SemaphoreType.DMA((S,)),
            pltpu.SemaphoreType.REGULAR,
            pltpu.SemaphoreType.REGULAR,
        ],
        compiler_params=pltpu.CompilerParams(collective_id=0),
    )(x, w_mat)


# device time: 769299 ns/iter; 1.9600x vs baseline; 1.9600x over previous
import numpy as np

import jax
import jax.numpy as jnp
from jax import lax
from jax.experimental import pallas as pl
from jax.experimental.pallas import tpu as pltpu

N_DEV = 32
S = 3
H_R = N_DEV // 2
H_L = N_DEV - 1 - H_R


def _ring_permutation():
    try:
        import distributed_mesh_v7x as dm

        mesh = dm.get_mesh("i", world_size=N_DEV)
        devs = list(np.asarray(mesh.devices).ravel())
        coords = [tuple(d.coords) for d in devs]
        coord_to_logical = {c: i for i, c in enumerate(coords)}
        xs = sorted({c[0] for c in coords})
        ys = sorted({c[1] for c in coords})
        zs = sorted({c[2] for c in coords})
        if len(xs) != 2 or len(xs) * len(ys) * len(zs) != N_DEV:
            return list(range(N_DEV))
        yz = []
        for zi, z in enumerate(zs):
            row = ys if zi % 2 == 0 else ys[::-1]
            yz.extend((y, z) for y in row)
        cycle = [(xs[0], y, z) for (y, z) in yz]
        cycle += [(xs[1], y, z) for (y, z) in reversed(yz)]
        for i in range(N_DEV):
            a, b = cycle[i], cycle[(i + 1) % N_DEV]
            if sum(abs(p - q) for p, q in zip(a, b)) != 1:
                return list(range(N_DEV))
        return [coord_to_logical[c] for c in cycle]
    except Exception:
        return list(range(N_DEV))


def kernel(x, w_mat):
    m_per, k = x.shape
    _, n_per = w_mat.shape
    assert m_per * N_DEV == 8192 and k == 8192

    ring = _ring_permutation()
    inv = np.argsort(np.asarray(ring))
    ring_arr = jnp.asarray(ring, dtype=jnp.int32)
    inv_arr = jnp.asarray(inv, dtype=jnp.int32)

    me = lax.axis_index("i")
    r = inv_arr[me]
    left = ring_arr[jnp.mod(r - 1, N_DEV)]
    right = ring_arr[jnp.mod(r + 1, N_DEV)]
    nbrs = jnp.stack([left, right]).astype(jnp.int32)
    origins_r = ring_arr[jnp.mod(r - jnp.arange(1, H_R + 1), N_DEV)]
    origins_l = ring_arr[jnp.mod(r + jnp.arange(1, H_L + 1), N_DEV)]

    def body(nbrs_ref, org_r_ref, org_l_ref, x_ref, w_ref, out_ref,
             xb, wb, buf_r, buf_l,
             send_r, recv_r, send_l, recv_l, credit_r, credit_l):
        my = lax.axis_index("i")
        lft = nbrs_ref[0]
        rgt = nbrs_ref[1]

        xb[...] = x_ref[...].astype(jnp.bfloat16)
        wb[...] = w_ref[...].astype(jnp.bfloat16)

        barrier = pltpu.get_barrier_semaphore()
        pl.semaphore_signal(barrier, inc=1, device_id=(lft,),
                            device_id_type=pl.DeviceIdType.MESH)
        pl.semaphore_signal(barrier, inc=1, device_id=(rgt,),
                            device_id_type=pl.DeviceIdType.MESH)
        pl.semaphore_wait(barrier, 2)

        def start_send(bufs, send_sems, recv_sems, dst_dev, h):
            s = (h - 1) % S
            src = xb if h == 1 else bufs.at[(h - 2) % S]
            desc = pltpu.make_async_remote_copy(
                src_ref=src,
                dst_ref=bufs.at[s],
                send_sem=send_sems.at[s],
                recv_sem=recv_sems.at[s],
                device_id=(dst_dev,),
                device_id_type=pl.DeviceIdType.MESH,
            )
            desc.start()
            return desc

        def gemm_store(chunk, origin):
            y = jnp.dot(chunk, wb[...], preferred_element_type=jnp.float32)
            out_ref[pl.ds(origin * m_per, m_per), :] = y * jax.nn.sigmoid(y)

        for h in range(1, H_R + 1):
            if h > S:
                pl.semaphore_wait(credit_r, 1)
            desc_r = start_send(buf_r, send_r, recv_r, rgt, h)
            desc_l = None
            if h <= H_L:
                if h > S:
                    pl.semaphore_wait(credit_l, 1)
                desc_l = start_send(buf_l, send_l, recv_l, lft, h)

            if h == 1:
                gemm_store(xb[...], my)
            else:
                ps = (h - 2) % S
                gemm_store(buf_r[ps], org_r_ref[h - 2])
                if h - 1 <= H_L:
                    gemm_store(buf_l[ps], org_l_ref[h - 2])

            desc_r.wait_send()
            if 2 <= h <= H_R - S + 1:
                pl.semaphore_signal(credit_r, inc=1, device_id=(lft,),
                                    device_id_type=pl.DeviceIdType.MESH)
            if desc_l is not None:
                desc_l.wait_send()
                if 2 <= h <= H_L - S + 1:
                    pl.semaphore_signal(credit_l, inc=1, device_id=(rgt,),
                                        device_id_type=pl.DeviceIdType.MESH)
            desc_r.wait_recv()
            if desc_l is not None:
                desc_l.wait_recv()

        gemm_store(buf_r[(H_R - 1) % S], org_r_ref[H_R - 1])
        gemm_store(buf_l[(H_L - 1) % S], org_l_ref[H_L - 1])

    return pl.pallas_call(
        body,
        out_shape=jax.ShapeDtypeStruct((N_DEV * m_per, n_per), jnp.float32),
        in_specs=[pl.BlockSpec(memory_space=pltpu.SMEM),
                  pl.BlockSpec(memory_space=pltpu.SMEM),
                  pl.BlockSpec(memory_space=pltpu.SMEM),
                  pl.BlockSpec(memory_space=pltpu.VMEM),
                  pl.BlockSpec(memory_space=pltpu.VMEM)],
        out_specs=pl.BlockSpec(memory_space=pltpu.VMEM),
        scratch_shapes=[
            pltpu.VMEM((m_per, k), jnp.bfloat16),
            pltpu.VMEM((k, n_per), jnp.bfloat16),
            pltpu.VMEM((S, m_per, k), jnp.bfloat16),
            pltpu.VMEM((S, m_per, k), jnp.bfloat16),
            pltpu.SemaphoreType.DMA((S,)),
            pltpu.SemaphoreType.DMA((S,)),
            pltpu.SemaphoreType.DMA((S,)),
            pltpu.SemaphoreType.DMA((S,)),
            pltpu.SemaphoreType.REGULAR,
            pltpu.SemaphoreType.REGULAR,
        ],
        compiler_params=pltpu.CompilerParams(collective_id=0),
    )(nbrs, origins_r, origins_l, x, w_mat)
